# baseline (speedup 1.0000x reference)
import functools

import jax
import jax.numpy as jnp
import numpy as np
from jax.experimental import pallas as pl
from jax.experimental.pallas import tpu as pltpu
from jax.sharding import Mesh, PartitionSpec as P

_NCOLS = 8


def _round_up(x, m):
    return (x + m - 1) // m * m


def _poincare_gather_kernel(eps, tab_ref, l_ref, r_ref, out_ref):
    tn = out_ref.shape[1]

    for sg in range(tn // 128):
        b = sg * 128
        rows_u = []
        rows_v = []
        for k in range(128):
            rows_u.append(tab_ref[l_ref[0, b + k]])
            rows_v.append(tab_ref[r_ref[0, b + k]])
        u = jnp.concatenate(rows_u, axis=0)
        v = jnp.concatenate(rows_v, axis=0)
        ut = jnp.transpose(u, (1, 0))
        vt = jnp.transpose(v, (1, 0))

        uu = jnp.sum(ut * ut, axis=0, keepdims=True)
        uv = jnp.sum(ut * vt, axis=0, keepdims=True)
        vv = jnp.sum(vt * vt, axis=0, keepdims=True)

        alpha = 1.0 - uu
        alpha = jnp.where(alpha <= 0.0, eps, alpha)
        beta = 1.0 - vv
        beta = jnp.where(beta <= 0.0, eps, beta)
        gamma = 1.0 + 2.0 * (uu - 2.0 * uv + vv) / (alpha * beta)
        gamma = jnp.maximum(gamma, 1.0)
        dist = jnp.log(gamma + jnp.sqrt(gamma * gamma - 1.0))

        out_ref[:, b:b + 128] = jnp.concatenate(
            [uu, uv, vv, alpha, beta, gamma, dist, jnp.zeros_like(uu)],
            axis=0)


def kernel(embeddings, left_idx, right_idx):
    eps = 1e-5
    emb = embeddings.astype(jnp.float32)
    vocab, d = emb.shape
    n = int(left_idx.shape[0])

    d_pad = _round_up(d, 128)
    tab = jnp.zeros((vocab + 16, 1, d_pad), jnp.float32)
    tab = tab.at[8:8 + vocab, 0, :d].set(emb)

    tn = 4096
    n_pad = _round_up(n, tn)
    li = jnp.pad(left_idx.astype(jnp.int32) + 8, (0, n_pad - n),
                 constant_values=8).reshape(1, n_pad)
    ri = jnp.pad(right_idx.astype(jnp.int32) + 8, (0, n_pad - n),
                 constant_values=8).reshape(1, n_pad)

    def run(tab_in, li_in, ri_in):
        n_loc = li_in.shape[1]
        return pl.pallas_call(
            functools.partial(_poincare_gather_kernel, float(eps)),
            out_shape=jax.ShapeDtypeStruct((_NCOLS, n_loc), jnp.float32),
            grid=(n_loc // tn,),
            in_specs=[
                pl.BlockSpec((vocab + 16, 1, d_pad), lambda i: (0, 0, 0)),
                pl.BlockSpec((1, tn), lambda i: (0, i),
                             memory_space=pltpu.SMEM),
                pl.BlockSpec((1, tn), lambda i: (0, i),
                             memory_space=pltpu.SMEM),
            ],
            out_specs=pl.BlockSpec((_NCOLS, tn), lambda i: (0, i)),
            compiler_params=pltpu.CompilerParams(
                dimension_semantics=("parallel",),
                vmem_limit_bytes=32 * 1024 * 1024),
        )(tab_in, li_in, ri_in)

    devs = jax.devices()
    if len(devs) >= 2 and (n_pad // tn) % 2 == 0:
        mesh = Mesh(np.asarray(devs[:2]), ("x",))
        packed = jax.shard_map(
            run, mesh=mesh,
            in_specs=(P(None, None, None), P(None, "x"), P(None, "x")),
            out_specs=P(None, "x"), check_vma=False,
        )(tab, li, ri)
    else:
        packed = run(tab, li, ri)

    rows = [packed[r, :n] for r in range(7)]
    uu, uv, vv, alpha, beta, gamma, dist = rows
    return (uu, uv, vv, alpha, beta, gamma), dist
